# baseline (speedup 1.0000x reference)
import functools

import jax
import jax.numpy as jnp
from jax import lax
from jax.experimental import pallas as pl
from jax.experimental.pallas import tpu as pltpu
from jax.experimental.pallas import tpu_sc as plsc

_N = 16384
_M = 128
_RES = 10000
_NBINS = _RES - 1
_NBP = 10000
_NC = 2
_NS = 16
_NW = _NC * _NS
_CPW = _M // _NW
_S = 1.0 / (_N + 1.0)


def _tc_bin_body(x_ref, o_ref):
    xb = x_ref[...]
    xc = jnp.clip(xb, 1.0 / _RES, 1.0 - 1.0 / _RES)
    ints = (jnp.round(xc * _RES) - 1.0).astype(jnp.int32)
    o_ref[...] = ints.T


_tc_bin = pl.pallas_call(
    _tc_bin_body,
    grid=(128,),
    in_specs=[pl.BlockSpec((_N // 128, _M), lambda i: (i, 0))],
    out_specs=pl.BlockSpec((_M, _N // 128), lambda i: (0, i)),
    out_shape=jax.ShapeDtypeStruct((_M, _N), jnp.int32),
)


def _sc_body(bins_hbm, lut_hbm, out_hbm, lut_v, col_a, col_b, hist_v, loss_v,
             sem_a, sem_b):
    wid = lax.axis_index("s") * _NC + lax.axis_index("c")
    pltpu.sync_copy(lut_hbm, lut_v)

    zeros = jnp.zeros((16,), jnp.float32)
    ones = jnp.ones((16,), jnp.float32)

    @plsc.parallel_loop(0, _NBP // 16, unroll=8)
    def _(j):
        hist_v[pl.ds(j * 16, 16)] = zeros

    lanes = lax.iota(jnp.int32, 16)
    fifteens = jnp.full((16,), 15, jnp.int32)
    loss_vec = zeros

    bufs = [col_a, col_b]
    sems = [sem_a, sem_b]
    copy = pltpu.async_copy(bins_hbm.at[wid * _CPW], col_a, sem_a)

    for k in range(_CPW):
        col_v = bufs[k % 2]
        if k + 1 < _CPW:
            next_copy = pltpu.async_copy(
                bins_hbm.at[wid * _CPW + k + 1], bufs[(k + 1) % 2],
                sems[(k + 1) % 2])
        copy.wait()

        @plsc.parallel_loop(0, _N // 16, unroll=8)
        def _(j):
            idx = col_v[pl.ds(j * 16, 16)]
            plsc.addupdate_scatter(hist_v, [idx], ones)

        @plsc.parallel_loop(0, _NBP // 16, unroll=4, carry=(zeros, zeros))
        def scan_out(j, carry):
            run, acc = carry
            sl = pl.ds(j * 16, 16)
            cnt = hist_v[sl]
            hist_v[sl] = zeros
            incl = plsc.cumsum(cnt)
            a = (incl - cnt) + run
            run = run + lax.gather(
                incl, fifteens[:, None],
                lax.GatherDimensionNumbers(
                    offset_dims=(), collapsed_slice_dims=(0,),
                    start_index_map=(0,)),
                slice_sizes=(1,),
                mode=lax.GatherScatterMode.PROMISE_IN_BOUNDS)
            v = lut_v[sl]
            d = a * _S - v
            u = cnt + 1.0
            t = d + _S * u
            contrib = cnt * (d * t + (_S * _S / 6.0) * (u * (cnt + u)))
            return run, acc + contrib

        _, acc = scan_out
        colloss = jnp.sum(acc)
        loss_vec = loss_vec + jnp.where(lanes == k, colloss, 0.0)
        if k + 1 < _CPW:
            copy = next_copy

    loss_v[...] = loss_vec
    pltpu.sync_copy(loss_v, out_hbm.at[wid])


_sc_loss = functools.partial(
    pl.kernel,
    out_type=jax.ShapeDtypeStruct((_NW, 16), jnp.float32),
    mesh=plsc.VectorSubcoreMesh(core_axis_name="c", subcore_axis_name="s"),
    scratch_types=[
        pltpu.VMEM((_NBP,), jnp.float32),
        pltpu.VMEM((_N,), jnp.int32),
        pltpu.VMEM((_N,), jnp.int32),
        pltpu.VMEM((_NBP,), jnp.float32),
        pltpu.VMEM((16,), jnp.float32),
        pltpu.SemaphoreType.DMA,
        pltpu.SemaphoreType.DMA,
    ],
    compiler_params=pltpu.CompilerParams(needs_layout_passes=False),
)(_sc_body)


def kernel(x_m, gamma, beta_pdf_lut, beta_cdf_lut):
    lut = jnp.pad(beta_cdf_lut[:, 0], (0, _NBP - _NBINS))
    bins_t = _tc_bin(x_m)
    parts = _sc_loss(bins_t, lut)
    return gamma * (jnp.sum(parts) * (1.0 / _M))


# speedup vs baseline: 2.1698x; 2.1698x over previous
import functools

import jax
import jax.numpy as jnp
from jax import lax
from jax.experimental import pallas as pl
from jax.experimental.pallas import tpu as pltpu
from jax.experimental.pallas import tpu_sc as plsc

_N = 16384
_M = 128
_RES = 10000
_NBINS = _RES - 1
_NBP = 10000
_NC = 2
_NS = 16
_NW = _NC * _NS
_CPW = _M // _NW
_S = 1.0 / (_N + 1.0)


def _tc_bin_body(x_ref, o_ref):
    xb = x_ref[...]
    xc = jnp.clip(xb, 1.0 / _RES, 1.0 - 1.0 / _RES)
    ints = (jnp.round(xc * _RES) - 1.0).astype(jnp.int32)
    o_ref[...] = ints.T


_tc_bin = pl.pallas_call(
    _tc_bin_body,
    grid=(8,),
    in_specs=[pl.BlockSpec((_N // 8, _M), lambda i: (i, 0))],
    out_specs=pl.BlockSpec((_M, _N // 8), lambda i: (0, i)),
    out_shape=jax.ShapeDtypeStruct((_M, _N), jnp.int32),
)


def _sc_body(bins_hbm, lut_hbm, out_hbm, lut_v, col_a, col_b, hist_v, loss_v,
             sem_a, sem_b):
    wid = lax.axis_index("s") * _NC + lax.axis_index("c")
    pltpu.sync_copy(lut_hbm, lut_v)

    zeros = jnp.zeros((16,), jnp.float32)
    ones = jnp.ones((16,), jnp.float32)

    @plsc.parallel_loop(0, _NBP // 16, unroll=8)
    def _(j):
        hist_v[pl.ds(j * 16, 16)] = zeros

    lanes = lax.iota(jnp.int32, 16)
    fifteens = jnp.full((16,), 15, jnp.int32)
    loss_vec = zeros

    bufs = [col_a, col_b]
    sems = [sem_a, sem_b]
    copy = pltpu.async_copy(bins_hbm.at[wid * _CPW], col_a, sem_a)

    for k in range(_CPW):
        col_v = bufs[k % 2]
        if k + 1 < _CPW:
            next_copy = pltpu.async_copy(
                bins_hbm.at[wid * _CPW + k + 1], bufs[(k + 1) % 2],
                sems[(k + 1) % 2])
        copy.wait()

        @plsc.parallel_loop(0, _N // 16, unroll=8)
        def _(j):
            idx = col_v[pl.ds(j * 16, 16)]
            plsc.addupdate_scatter(hist_v, [idx], ones)

        @plsc.parallel_loop(0, _NBP // 16, unroll=4, carry=(zeros, zeros))
        def scan_out(j, carry):
            run, acc = carry
            sl = pl.ds(j * 16, 16)
            cnt = hist_v[sl]
            hist_v[sl] = zeros
            incl = plsc.cumsum(cnt)
            a = (incl - cnt) + run
            run = run + lax.gather(
                incl, fifteens[:, None],
                lax.GatherDimensionNumbers(
                    offset_dims=(), collapsed_slice_dims=(0,),
                    start_index_map=(0,)),
                slice_sizes=(1,),
                mode=lax.GatherScatterMode.PROMISE_IN_BOUNDS)
            v = lut_v[sl]
            d = a * _S - v
            u = cnt + 1.0
            t = d + _S * u
            contrib = cnt * (d * t + (_S * _S / 6.0) * (u * (cnt + u)))
            return run, acc + contrib

        _, acc = scan_out
        colloss = jnp.sum(acc)
        loss_vec = loss_vec + jnp.where(lanes == k, colloss, 0.0)
        if k + 1 < _CPW:
            copy = next_copy

    loss_v[...] = loss_vec
    pltpu.sync_copy(loss_v, out_hbm.at[wid])


_sc_loss = functools.partial(
    pl.kernel,
    out_type=jax.ShapeDtypeStruct((_NW, 16), jnp.float32),
    mesh=plsc.VectorSubcoreMesh(core_axis_name="c", subcore_axis_name="s"),
    scratch_types=[
        pltpu.VMEM((_NBP,), jnp.float32),
        pltpu.VMEM((_N,), jnp.int32),
        pltpu.VMEM((_N,), jnp.int32),
        pltpu.VMEM((_NBP,), jnp.float32),
        pltpu.VMEM((16,), jnp.float32),
        pltpu.SemaphoreType.DMA,
        pltpu.SemaphoreType.DMA,
    ],
    compiler_params=pltpu.CompilerParams(needs_layout_passes=False),
)(_sc_body)


def kernel(x_m, gamma, beta_pdf_lut, beta_cdf_lut):
    lut = jnp.pad(beta_cdf_lut[:, 0], (0, _NBP - _NBINS))
    bins_t = _tc_bin(x_m)
    parts = _sc_loss(bins_t, lut)
    return gamma * (jnp.sum(parts) * (1.0 / _M))


# speedup vs baseline: 2.2735x; 1.0478x over previous
import functools

import jax
import jax.numpy as jnp
from jax import lax
from jax.experimental import pallas as pl
from jax.experimental.pallas import tpu as pltpu
from jax.experimental.pallas import tpu_sc as plsc

_N = 16384
_M = 128
_RES = 10000
_NBINS = _RES - 1
_NBP = 10000
_NC = 2
_NS = 16
_NW = _NC * _NS
_CPW = _M // _NW
_S = 1.0 / (_N + 1.0)


def _tc_bin_body(x_ref, o_ref):
    xb = x_ref[...]
    xc = jnp.clip(xb, 1.0 / _RES, 1.0 - 1.0 / _RES)
    ints = (jnp.round(xc * _RES) - 1.0).astype(jnp.int32)
    o_ref[...] = ints.T


_tc_bin = pl.pallas_call(
    _tc_bin_body,
    grid=(4,),
    in_specs=[pl.BlockSpec((_N // 4, _M), lambda i: (i, 0))],
    out_specs=pl.BlockSpec((_M, _N // 4), lambda i: (0, i)),
    out_shape=jax.ShapeDtypeStruct((_M, _N), jnp.int32),
)


def _sc_body(bins_hbm, lut_hbm, out_hbm, lut_v, col_a, col_b, hist_v, loss_v,
             sem_a, sem_b):
    wid = lax.axis_index("s") * _NC + lax.axis_index("c")
    pltpu.sync_copy(lut_hbm, lut_v)

    zeros = jnp.zeros((16,), jnp.float32)
    ones = jnp.ones((16,), jnp.float32)

    @plsc.parallel_loop(0, _NBP // 16, unroll=8)
    def _(j):
        hist_v[pl.ds(j * 16, 16)] = zeros

    lanes = lax.iota(jnp.int32, 16)
    fifteens = jnp.full((16,), 15, jnp.int32)
    loss_vec = zeros

    bufs = [col_a, col_b]
    sems = [sem_a, sem_b]
    copy = pltpu.async_copy(bins_hbm.at[wid * _CPW], col_a, sem_a)

    for k in range(_CPW):
        col_v = bufs[k % 2]
        if k + 1 < _CPW:
            next_copy = pltpu.async_copy(
                bins_hbm.at[wid * _CPW + k + 1], bufs[(k + 1) % 2],
                sems[(k + 1) % 2])
        copy.wait()

        @plsc.parallel_loop(0, _N // 16, unroll=8)
        def _(j):
            idx = col_v[pl.ds(j * 16, 16)]
            plsc.addupdate_scatter(hist_v, [idx], ones)

        @plsc.parallel_loop(0, _NBP // 16, unroll=4, carry=(zeros, zeros))
        def scan_out(j, carry):
            run, acc = carry
            sl = pl.ds(j * 16, 16)
            cnt = hist_v[sl]
            hist_v[sl] = zeros
            incl = plsc.cumsum(cnt)
            a = (incl - cnt) + run
            run = run + lax.gather(
                incl, fifteens[:, None],
                lax.GatherDimensionNumbers(
                    offset_dims=(), collapsed_slice_dims=(0,),
                    start_index_map=(0,)),
                slice_sizes=(1,),
                mode=lax.GatherScatterMode.PROMISE_IN_BOUNDS)
            v = lut_v[sl]
            d = a * _S - v
            u = cnt + 1.0
            t = d + _S * u
            contrib = cnt * (d * t + (_S * _S / 6.0) * (u * (cnt + u)))
            return run, acc + contrib

        _, acc = scan_out
        colloss = jnp.sum(acc)
        loss_vec = loss_vec + jnp.where(lanes == k, colloss, 0.0)
        if k + 1 < _CPW:
            copy = next_copy

    loss_v[...] = loss_vec
    pltpu.sync_copy(loss_v, out_hbm.at[wid])


_sc_loss = functools.partial(
    pl.kernel,
    out_type=jax.ShapeDtypeStruct((_NW, 16), jnp.float32),
    mesh=plsc.VectorSubcoreMesh(core_axis_name="c", subcore_axis_name="s"),
    scratch_types=[
        pltpu.VMEM((_NBP,), jnp.float32),
        pltpu.VMEM((_N,), jnp.int32),
        pltpu.VMEM((_N,), jnp.int32),
        pltpu.VMEM((_NBP,), jnp.float32),
        pltpu.VMEM((16,), jnp.float32),
        pltpu.SemaphoreType.DMA,
        pltpu.SemaphoreType.DMA,
    ],
    compiler_params=pltpu.CompilerParams(needs_layout_passes=False),
)(_sc_body)


def kernel(x_m, gamma, beta_pdf_lut, beta_cdf_lut):
    lut = jnp.pad(beta_cdf_lut[:, 0], (0, _NBP - _NBINS))
    bins_t = _tc_bin(x_m)
    parts = _sc_loss(bins_t, lut)
    return gamma * (jnp.sum(parts) * (1.0 / _M))
